# baseline (speedup 1.0000x reference)
import functools

import jax
import jax.numpy as jnp
from jax import lax
from jax.experimental import pallas as pl
from jax.experimental.pallas import tpu as pltpu
from jax.experimental.pallas import tpu_sc as plsc

E = 8
TOP_K = 2
D = 768
H = 1024
N = 2048
T = 256
NB = 24
CAP = NB * T
H_CHUNK = 1024
NEG = -3.0e38

NW = 32
SLOTS_W = CAP // NW
TOK_W = N // NW
APW = (2 * N) // NW
HPW = APW // 2


def _routing_kernel(x_ref, ue_ref, onb_ref,
                    idx_ref, w_ref, post_ref, be_ref, bv_ref):
    x = x_ref[...]
    ue = ue_ref[...]
    onb = onb_ref[...]
    sims = lax.dot_general(x, ue, (((1,), (1,)), ((), ())))
    xn = jnp.sqrt(jnp.sum(x * x, axis=1, keepdims=True))
    en = jnp.sqrt(jnp.sum(ue * ue, axis=1, keepdims=True))
    tok_n = x / (xn + 1e-8)
    exp_n = ue / (en + 1e-8)
    cos = lax.dot_general(tok_n, exp_n, (((1,), (1,)), ((), ())))
    t = cos + 0.1 * onb
    eidx = lax.broadcasted_iota(jnp.int32, (N, E), 1)
    m = jnp.min(sims, axis=1, keepdims=True)
    excl = jnp.max(jnp.where(sims == m, eidx, -1), axis=1, keepdims=True)
    t = jnp.where(eidx == excl, NEG, t)
    t0 = jnp.max(t, axis=1, keepdims=True)
    i0 = jnp.min(jnp.where(t == t0, eidx, E), axis=1, keepdims=True)
    t1m = jnp.where(eidx == i0, NEG, t)
    t1 = jnp.max(t1m, axis=1, keepdims=True)
    i1 = jnp.min(jnp.where(t1m == t1, eidx, E), axis=1, keepdims=True)
    ed = jnp.exp(t1 - t0)
    s = 1.0 + ed
    w0 = 1.0 / s
    w1 = ed / s
    idx_ref[...] = jnp.concatenate([i0, i1], axis=1)
    w_ref[...] = jnp.concatenate([w0, w1], axis=1)

    a0 = jnp.where(eidx == i0, 1.0, 0.0)
    a1 = jnp.where(eidx == i1, 1.0, 0.0)
    a = jnp.concatenate([a0, a1], axis=1)
    cum = a
    k = 1
    while k < N:
        z = jnp.zeros((k, 2 * E), jnp.float32)
        cum = cum + jnp.concatenate([z, cum[: N - k]], axis=0)
        k *= 2
    cum = cum - a
    cum0 = cum[:, :E]
    cum1 = cum[:, E:]
    cnt0 = jnp.sum(a0, axis=0, keepdims=True)
    cnt1 = jnp.sum(a1, axis=0, keepdims=True)
    cnt = cnt0 + cnt1
    al = jnp.float32(T) * jnp.ceil(cnt * (1.0 / T))
    e8r = lax.broadcasted_iota(jnp.int32, (E, E), 0).astype(jnp.float32)
    e8c = lax.broadcasted_iota(jnp.int32, (E, E), 1).astype(jnp.float32)
    strict_lt = jnp.where(e8r < e8c, 1.0, 0.0)
    base = lax.dot_general(al, strict_lt, (((1,), (0,)), ((), ())))
    pos0 = jnp.sum(a0 * (base + cum0), axis=1, keepdims=True)
    pos1 = jnp.sum(a1 * (base + cnt0 + cum1), axis=1, keepdims=True)
    post_ref[...] = jnp.concatenate(
        [pos0.astype(jnp.int32).T, pos1.astype(jnp.int32).T], axis=0)

    bstart = jnp.float32(T) * lax.broadcasted_iota(jnp.int32, (NB, E), 0).astype(jnp.float32)
    basem = jnp.broadcast_to(base, (NB, E))
    alm = jnp.broadcast_to(al, (NB, E))
    sel = jnp.where((bstart >= basem) & (bstart < basem + alm), 1.0, 0.0)
    erow = lax.broadcasted_iota(jnp.int32, (NB, E), 1).astype(jnp.float32)
    be = jnp.sum(sel * erow, axis=1, keepdims=True)
    total = jnp.sum(al)
    valid = bstart[:, :1] < total
    e1r = lax.broadcasted_iota(jnp.int32, (1, E), 1).astype(jnp.float32)
    last_e = jnp.max(jnp.where(al > 0.0, e1r, -1.0))
    be_ref[...] = jnp.where(valid, be, last_e).astype(jnp.int32)
    bv_ref[...] = valid.astype(jnp.int32)


def _routing_call(x, ue, onb):
    return pl.pallas_call(
        _routing_kernel,
        in_specs=[
            pl.BlockSpec((N, D), lambda: (0, 0)),
            pl.BlockSpec((E, D), lambda: (0, 0)),
            pl.BlockSpec((1, E), lambda: (0, 0)),
        ],
        out_specs=[
            pl.BlockSpec((N, TOP_K), lambda: (0, 0)),
            pl.BlockSpec((N, TOP_K), lambda: (0, 0)),
            pl.BlockSpec((TOP_K, N), lambda: (0, 0)),
            pl.BlockSpec((NB, 1), lambda: (0, 0)),
            pl.BlockSpec((NB, 1), lambda: (0, 0)),
        ],
        out_shape=[
            jax.ShapeDtypeStruct((N, TOP_K), jnp.int32),
            jax.ShapeDtypeStruct((N, TOP_K), jnp.float32),
            jax.ShapeDtypeStruct((TOP_K, N), jnp.int32),
            jax.ShapeDtypeStruct((NB, 1), jnp.int32),
            jax.ShapeDtypeStruct((NB, 1), jnp.int32),
        ],
    )(x, ue, onb)


def _make_sc_dispatch():
    mesh = plsc.VectorSubcoreMesh(core_axis_name="c", subcore_axis_name="s", num_cores=2, num_subcores=16)

    @functools.partial(
        pl.kernel, mesh=mesh,
        compiler_params=pltpu.CompilerParams(needs_layout_passes=False),
        out_type=[
            jax.ShapeDtypeStruct((CAP, D), jnp.float32),
            jax.ShapeDtypeStruct((CAP,), jnp.float32),
        ],
        scratch_types=[
            pltpu.VMEM((HPW,), jnp.int32),
            pltpu.VMEM((HPW,), jnp.int32),
            pltpu.VMEM((APW,), jnp.float32),
            pltpu.VMEM((HPW, D), jnp.float32),
            pltpu.VMEM((HPW, D), jnp.float32),
            pltpu.SemaphoreType.DMA,
            pltpu.SemaphoreType.DMA,
            pltpu.SemaphoreType.DMA,
            pltpu.SemaphoreType.DMA,
            pltpu.SemaphoreType.DMA,
        ],
    )
    def dispatch(x_hbm, posr_hbm, wr_hbm, xs_hbm, ws_hbm,
                 pos_a, pos_b, w_v, rows_a, rows_b,
                 sem_a, sem_b, sem_w, sem_xa, sem_xb):
        wid = lax.axis_index("s") * 2 + lax.axis_index("c")
        tok0 = pl.multiple_of((wid * APW) & (N - 1), APW)
        c1 = pltpu.async_copy(posr_hbm.at[wid, 0], pos_a, sem_a)
        c2 = pltpu.async_copy(posr_hbm.at[wid, 1], pos_b, sem_b)
        c3 = pltpu.async_copy(wr_hbm.at[wid], w_v, sem_w)
        ca = pltpu.async_copy(x_hbm.at[pl.ds(tok0, HPW)], rows_a, sem_xa)
        cb = pltpu.async_copy(x_hbm.at[pl.ds(tok0 + HPW, HPW)], rows_b, sem_xb)
        c1.wait()
        ca.wait()
        sa = pltpu.async_copy(rows_a, xs_hbm.at[pos_a], sem_xa)
        c2.wait()
        cb.wait()
        sb = pltpu.async_copy(rows_b, xs_hbm.at[pos_b], sem_xb)
        c3.wait()
        s1 = pltpu.async_copy(w_v.at[pl.ds(0, HPW)], ws_hbm.at[pos_a], sem_a)
        s2 = pltpu.async_copy(w_v.at[pl.ds(HPW, HPW)], ws_hbm.at[pos_b], sem_b)
        sa.wait()
        sb.wait()
        s1.wait()
        s2.wait()

    return dispatch


def _ffn_kernel(be_ref, bv_ref, xs_ref, w1_ref, b1_ref, w2_ref, b2_ref,
                ws_ref, ys_ref):
    b = pl.program_id(0)

    @pl.when(bv_ref[b] != 0)
    def _():
        x = xs_ref[...]
        b2 = b2_ref[0]


        def body(c, y):
            h = lax.dot_general(
                x, w1_ref[0, :, pl.ds(c * H_CHUNK, H_CHUNK)],
                (((1,), (0,)), ((), ())), preferred_element_type=jnp.float32)
            h = h + b1_ref[0, :, pl.ds(c * H_CHUNK, H_CHUNK)]
            a = 0.5 * h * (1.0 + lax.erf(h * 0.7071067811865476))
            y = y + lax.dot_general(
                a, w2_ref[0, pl.ds(c * H_CHUNK, H_CHUNK), :],
                (((1,), (0,)), ((), ())), preferred_element_type=jnp.float32)
            return y

        y = lax.fori_loop(0, H // H_CHUNK, body, jnp.zeros((T, D), jnp.float32))
        ys_ref[...] = (y + b2) * ws_ref[0].T


def _ffn_call(xs, W1, b1r, W2, b2r, wsr, be, bv):
    grid_spec = pltpu.PrefetchScalarGridSpec(
        num_scalar_prefetch=2,
        grid=(NB,),
        in_specs=[
            pl.BlockSpec((T, D), lambda b, be_r, bv_r: (b, 0)),
            pl.BlockSpec((1, D, H), lambda b, be_r, bv_r: (be_r[b], 0, 0)),
            pl.BlockSpec((1, 1, H), lambda b, be_r, bv_r: (be_r[b], 0, 0)),
            pl.BlockSpec((1, H, D), lambda b, be_r, bv_r: (be_r[b], 0, 0)),
            pl.BlockSpec((1, 1, D), lambda b, be_r, bv_r: (be_r[b], 0, 0)),
            pl.BlockSpec((1, 1, T), lambda b, be_r, bv_r: (b, 0, 0)),
        ],
        out_specs=pl.BlockSpec((T, D), lambda b, be_r, bv_r: (b, 0)),
    )
    return pl.pallas_call(
        _ffn_kernel,
        grid_spec=grid_spec,
        out_shape=jax.ShapeDtypeStruct((CAP, D), jnp.float32),
    )(be, bv, xs, W1, b1r, W2, b2r, wsr)


def _make_sc_combine():
    mesh = plsc.VectorSubcoreMesh(core_axis_name="c", subcore_axis_name="s", num_cores=2, num_subcores=16)

    @functools.partial(
        pl.kernel, mesh=mesh,
        out_type=jax.ShapeDtypeStruct((N, D), jnp.float32),
        scratch_types=[
            pltpu.VMEM((TOK_W,), jnp.int32),
            pltpu.VMEM((TOK_W,), jnp.int32),
            pltpu.VMEM((TOK_W, D), jnp.float32),
            pltpu.VMEM((TOK_W, D), jnp.float32),
            pltpu.SemaphoreType.DMA,
            pltpu.SemaphoreType.DMA,
        ],
    )
    def combine(ys_hbm, p0_hbm, p1_hbm, out_hbm, p0v, p1v, buf0, buf1,
                sem0, sem1):
        wid = lax.axis_index("s") * 2 + lax.axis_index("c")
        base = wid * TOK_W
        c1 = pltpu.async_copy(p0_hbm.at[pl.ds(base, TOK_W)], p0v, sem0)
        c2 = pltpu.async_copy(p1_hbm.at[pl.ds(base, TOK_W)], p1v, sem1)
        c1.wait()
        c3 = pltpu.async_copy(ys_hbm.at[p0v], buf0, sem0)
        c2.wait()
        c4 = pltpu.async_copy(ys_hbm.at[p1v], buf1, sem1)
        c3.wait()
        c4.wait()

        @plsc.parallel_loop(0, TOK_W, unroll=2)
        def _(i):
            for k in range(D // 16):
                sl = pl.ds(k * 16, 16)
                buf0[i, sl] = buf0[i, sl] + buf1[i, sl]
        pltpu.sync_copy(buf0, out_hbm.at[pl.ds(base, TOK_W)])

    return combine


_SC_CACHE = {}


def _sc_kernels():
    if "dispatch" not in _SC_CACHE:
        _SC_CACHE["dispatch"] = _make_sc_dispatch()
        _SC_CACHE["combine"] = _make_sc_combine()
    return _SC_CACHE["dispatch"], _SC_CACHE["combine"]


@jax.jit
def kernel(token_embeddings, uzman_embeddings, W1, b1, W2, b2,
           onbellek_durumu):
    b, s, d = token_embeddings.shape
    x = token_embeddings.reshape(-1, d)
    onb = onbellek_durumu.reshape(1, E)
    b1r = b1.reshape(E, 1, H)
    b2r = b2.reshape(E, 1, D)

    sc_dispatch, sc_combine = _sc_kernels()
    idx, w, post, be, bv = _routing_call(x, uzman_embeddings, onb)
    posr = post.reshape(NW, 2, HPW)
    wr = jnp.transpose(w).reshape(NW, APW)
    xs, ws = sc_dispatch(x, posr, wr)
    wsr = ws.reshape(NB, 1, T)
    ys = _ffn_call(xs, W1, b1r, W2, b2r, wsr, be.reshape(NB),
                   bv.reshape(NB))
    out = sc_combine(ys, post[0], post[1])
    return (out.reshape(b, s, d), idx.reshape(b, s, TOP_K),
            w.reshape(b, s, TOP_K))


# speedup vs baseline: 1.0023x; 1.0023x over previous
import functools

import jax
import jax.numpy as jnp
from jax import lax
from jax.experimental import pallas as pl
from jax.experimental.pallas import tpu as pltpu
from jax.experimental.pallas import tpu_sc as plsc

E = 8
TOP_K = 2
D = 768
H = 1024
N = 2048
T = 256
NB = 24
CAP = NB * T
H_CHUNK = 1024
NEG = -3.0e38

NW = 32
SLOTS_W = CAP // NW
TOK_W = N // NW
APW = (2 * N) // NW
HPW = APW // 2


def _routing_kernel(x_ref, ue_ref, onb_ref,
                    idx_ref, w_ref, post_ref, be_ref, bv_ref):
    x = x_ref[...]
    ue = ue_ref[...]
    onb = onb_ref[...]
    sims = lax.dot_general(x, ue, (((1,), (1,)), ((), ())))
    xn = jnp.sqrt(jnp.sum(x * x, axis=1, keepdims=True))
    en = jnp.sqrt(jnp.sum(ue * ue, axis=1, keepdims=True))
    tok_n = x / (xn + 1e-8)
    exp_n = ue / (en + 1e-8)
    cos = lax.dot_general(tok_n, exp_n, (((1,), (1,)), ((), ())))
    t = cos + 0.1 * onb
    eidx = lax.broadcasted_iota(jnp.int32, (N, E), 1)
    m = jnp.min(sims, axis=1, keepdims=True)
    excl = jnp.max(jnp.where(sims == m, eidx, -1), axis=1, keepdims=True)
    t = jnp.where(eidx == excl, NEG, t)
    t0 = jnp.max(t, axis=1, keepdims=True)
    i0 = jnp.min(jnp.where(t == t0, eidx, E), axis=1, keepdims=True)
    t1m = jnp.where(eidx == i0, NEG, t)
    t1 = jnp.max(t1m, axis=1, keepdims=True)
    i1 = jnp.min(jnp.where(t1m == t1, eidx, E), axis=1, keepdims=True)
    ed = jnp.exp(t1 - t0)
    s = 1.0 + ed
    w0 = 1.0 / s
    w1 = ed / s
    idx_ref[...] = jnp.concatenate([i0, i1], axis=1)
    w_ref[...] = jnp.concatenate([w0, w1], axis=1)

    a0 = jnp.where(eidx == i0, 1.0, 0.0)
    a1 = jnp.where(eidx == i1, 1.0, 0.0)
    a = jnp.concatenate([a0, a1], axis=1)
    cum = a
    k = 1
    while k < N:
        z = jnp.zeros((k, 2 * E), jnp.float32)
        cum = cum + jnp.concatenate([z, cum[: N - k]], axis=0)
        k *= 2
    cum = cum - a
    cum0 = cum[:, :E]
    cum1 = cum[:, E:]
    cnt0 = jnp.sum(a0, axis=0, keepdims=True)
    cnt1 = jnp.sum(a1, axis=0, keepdims=True)
    cnt = cnt0 + cnt1
    al = jnp.float32(T) * jnp.ceil(cnt * (1.0 / T))
    e8r = lax.broadcasted_iota(jnp.int32, (E, E), 0).astype(jnp.float32)
    e8c = lax.broadcasted_iota(jnp.int32, (E, E), 1).astype(jnp.float32)
    strict_lt = jnp.where(e8r < e8c, 1.0, 0.0)
    base = lax.dot_general(al, strict_lt, (((1,), (0,)), ((), ())))
    pos0 = jnp.sum(a0 * (base + cum0), axis=1, keepdims=True)
    pos1 = jnp.sum(a1 * (base + cnt0 + cum1), axis=1, keepdims=True)
    post_ref[...] = jnp.concatenate(
        [pos0.astype(jnp.int32).T, pos1.astype(jnp.int32).T], axis=0)

    bstart = jnp.float32(T) * lax.broadcasted_iota(jnp.int32, (NB, E), 0).astype(jnp.float32)
    basem = jnp.broadcast_to(base, (NB, E))
    alm = jnp.broadcast_to(al, (NB, E))
    sel = jnp.where((bstart >= basem) & (bstart < basem + alm), 1.0, 0.0)
    erow = lax.broadcasted_iota(jnp.int32, (NB, E), 1).astype(jnp.float32)
    be = jnp.sum(sel * erow, axis=1, keepdims=True)
    total = jnp.sum(al)
    valid = bstart[:, :1] < total
    e1r = lax.broadcasted_iota(jnp.int32, (1, E), 1).astype(jnp.float32)
    last_e = jnp.max(jnp.where(al > 0.0, e1r, -1.0))
    be_ref[...] = jnp.where(valid, be, last_e).astype(jnp.int32)
    bv_ref[...] = valid.astype(jnp.int32)


def _routing_call(x, ue, onb):
    return pl.pallas_call(
        _routing_kernel,
        in_specs=[
            pl.BlockSpec((N, D), lambda: (0, 0)),
            pl.BlockSpec((E, D), lambda: (0, 0)),
            pl.BlockSpec((1, E), lambda: (0, 0)),
        ],
        out_specs=[
            pl.BlockSpec((N, TOP_K), lambda: (0, 0)),
            pl.BlockSpec((N, TOP_K), lambda: (0, 0)),
            pl.BlockSpec((TOP_K, N), lambda: (0, 0)),
            pl.BlockSpec((NB, 1), lambda: (0, 0)),
            pl.BlockSpec((NB, 1), lambda: (0, 0)),
        ],
        out_shape=[
            jax.ShapeDtypeStruct((N, TOP_K), jnp.int32),
            jax.ShapeDtypeStruct((N, TOP_K), jnp.float32),
            jax.ShapeDtypeStruct((TOP_K, N), jnp.int32),
            jax.ShapeDtypeStruct((NB, 1), jnp.int32),
            jax.ShapeDtypeStruct((NB, 1), jnp.int32),
        ],
    )(x, ue, onb)


def _make_sc_dispatch():
    mesh = plsc.VectorSubcoreMesh(core_axis_name="c", subcore_axis_name="s", num_cores=2, num_subcores=16)

    @functools.partial(
        pl.kernel, mesh=mesh,
        compiler_params=pltpu.CompilerParams(needs_layout_passes=False),
        out_type=[
            jax.ShapeDtypeStruct((CAP, D), jnp.float32),
            jax.ShapeDtypeStruct((CAP,), jnp.float32),
        ],
        scratch_types=[
            pltpu.VMEM((HPW,), jnp.int32),
            pltpu.VMEM((HPW,), jnp.int32),
            pltpu.VMEM((APW,), jnp.float32),
            pltpu.VMEM((HPW, D), jnp.float32),
            pltpu.VMEM((HPW, D), jnp.float32),
            pltpu.SemaphoreType.DMA,
            pltpu.SemaphoreType.DMA,
            pltpu.SemaphoreType.DMA,
            pltpu.SemaphoreType.DMA,
            pltpu.SemaphoreType.DMA,
        ],
    )
    def dispatch(x_hbm, posr_hbm, wr_hbm, xs_hbm, ws_hbm,
                 pos_a, pos_b, w_v, rows_a, rows_b,
                 sem_a, sem_b, sem_w, sem_xa, sem_xb):
        wid = lax.axis_index("s") * 2 + lax.axis_index("c")
        tok0 = pl.multiple_of((wid * APW) & (N - 1), APW)
        c1 = pltpu.async_copy(posr_hbm.at[wid, 0], pos_a, sem_a)
        c2 = pltpu.async_copy(posr_hbm.at[wid, 1], pos_b, sem_b)
        c3 = pltpu.async_copy(wr_hbm.at[wid], w_v, sem_w)
        ca = pltpu.async_copy(x_hbm.at[pl.ds(tok0, HPW)], rows_a, sem_xa)
        cb = pltpu.async_copy(x_hbm.at[pl.ds(tok0 + HPW, HPW)], rows_b, sem_xb)
        c1.wait()
        ca.wait()
        sa = pltpu.async_copy(rows_a, xs_hbm.at[pos_a], sem_xa)
        c2.wait()
        cb.wait()
        sb = pltpu.async_copy(rows_b, xs_hbm.at[pos_b], sem_xb)
        c3.wait()
        s1 = pltpu.async_copy(w_v.at[pl.ds(0, HPW)], ws_hbm.at[pos_a], sem_a)
        s2 = pltpu.async_copy(w_v.at[pl.ds(HPW, HPW)], ws_hbm.at[pos_b], sem_b)
        sa.wait()
        sb.wait()
        s1.wait()
        s2.wait()

    return dispatch


def _ffn_kernel(be_ref, bv_ref, xs_ref, w1_ref, b1_ref, w2_ref, b2_ref,
                ws_ref, ys_ref):
    b = pl.program_id(0)

    @pl.when(bv_ref[b] != 0)
    def _():
        x = xs_ref[...]
        b2 = b2_ref[0]


        def body(c, y):
            h = lax.dot_general(
                x, w1_ref[0, :, pl.ds(c * H_CHUNK, H_CHUNK)],
                (((1,), (0,)), ((), ())), preferred_element_type=jnp.float32)
            h = h + b1_ref[0, :, pl.ds(c * H_CHUNK, H_CHUNK)]
            a = 0.5 * h * (1.0 + lax.erf(h * 0.7071067811865476))
            y = y + lax.dot_general(
                a, w2_ref[0, pl.ds(c * H_CHUNK, H_CHUNK), :],
                (((1,), (0,)), ((), ())), preferred_element_type=jnp.float32)
            return y

        y = lax.fori_loop(0, H // H_CHUNK, body, jnp.zeros((T, D), jnp.float32))
        ys_ref[...] = (y + b2) * ws_ref[0].T


def _ffn_call(xs, W1, b1r, W2, b2r, wsr, be, bv):
    grid_spec = pltpu.PrefetchScalarGridSpec(
        num_scalar_prefetch=2,
        grid=(NB,),
        in_specs=[
            pl.BlockSpec((T, D), lambda b, be_r, bv_r: (b, 0)),
            pl.BlockSpec((1, D, H), lambda b, be_r, bv_r: (be_r[b], 0, 0)),
            pl.BlockSpec((1, 1, H), lambda b, be_r, bv_r: (be_r[b], 0, 0)),
            pl.BlockSpec((1, H, D), lambda b, be_r, bv_r: (be_r[b], 0, 0)),
            pl.BlockSpec((1, 1, D), lambda b, be_r, bv_r: (be_r[b], 0, 0)),
            pl.BlockSpec((1, 1, T), lambda b, be_r, bv_r: (b, 0, 0)),
        ],
        out_specs=pl.BlockSpec((T, D), lambda b, be_r, bv_r: (b, 0)),
    )
    return pl.pallas_call(
        _ffn_kernel,
        grid_spec=grid_spec,
        out_shape=jax.ShapeDtypeStruct((CAP, D), jnp.float32),
    )(be, bv, xs, W1, b1r, W2, b2r, wsr)


def _make_sc_combine():
    mesh = plsc.VectorSubcoreMesh(core_axis_name="c", subcore_axis_name="s", num_cores=2, num_subcores=16)

    @functools.partial(
        pl.kernel, mesh=mesh,
        out_type=jax.ShapeDtypeStruct((N, D), jnp.float32),
        scratch_types=[
            pltpu.VMEM((TOK_W,), jnp.int32),
            pltpu.VMEM((TOK_W,), jnp.int32),
            pltpu.VMEM((TOK_W, D), jnp.float32),
            pltpu.VMEM((TOK_W, D), jnp.float32),
            pltpu.SemaphoreType.DMA,
            pltpu.SemaphoreType.DMA,
        ],
    )
    def combine(ys_hbm, p0_hbm, p1_hbm, out_hbm, p0v, p1v, buf0, buf1,
                sem0, sem1):
        wid = lax.axis_index("s") * 2 + lax.axis_index("c")
        base = wid * TOK_W
        c1 = pltpu.async_copy(p0_hbm.at[pl.ds(base, TOK_W)], p0v, sem0)
        c2 = pltpu.async_copy(p1_hbm.at[pl.ds(base, TOK_W)], p1v, sem1)
        c1.wait()
        c3 = pltpu.async_copy(ys_hbm.at[p0v], buf0, sem0)
        c2.wait()
        c4 = pltpu.async_copy(ys_hbm.at[p1v], buf1, sem1)
        c3.wait()
        c4.wait()

        def abody(i, carry):
            for k in range(D // 16):
                sl = pl.ds(k * 16, 16)
                buf0[i, sl] = buf0[i, sl] + buf1[i, sl]
            return carry

        lax.fori_loop(0, TOK_W, abody, 0)
        pltpu.sync_copy(buf0, out_hbm.at[pl.ds(base, TOK_W)])

    return combine


_SC_CACHE = {}


def _sc_kernels():
    if "dispatch" not in _SC_CACHE:
        _SC_CACHE["dispatch"] = _make_sc_dispatch()
        _SC_CACHE["combine"] = _make_sc_combine()
    return _SC_CACHE["dispatch"], _SC_CACHE["combine"]


@jax.jit
def kernel(token_embeddings, uzman_embeddings, W1, b1, W2, b2,
           onbellek_durumu):
    b, s, d = token_embeddings.shape
    x = token_embeddings.reshape(-1, d)
    onb = onbellek_durumu.reshape(1, E)
    b1r = b1.reshape(E, 1, H)
    b2r = b2.reshape(E, 1, D)

    sc_dispatch, sc_combine = _sc_kernels()
    idx, w, post, be, bv = _routing_call(x, uzman_embeddings, onb)
    posr = post.reshape(NW, 2, HPW)
    wr = jnp.transpose(w).reshape(NW, APW)
    xs, ws = sc_dispatch(x, posr, wr)
    wsr = ws.reshape(NB, 1, T)
    ys = _ffn_call(xs, W1, b1r, W2, b2r, wsr, be.reshape(NB),
                   bv.reshape(NB))
    out = sc_combine(ys, post[0], post[1])
    return (out.reshape(b, s, d), idx.reshape(b, s, TOP_K),
            w.reshape(b, s, TOP_K))
